# Initial kernel scaffold; baseline (speedup 1.0000x reference)
import jax
import jax.numpy as jnp
from jax.experimental import pallas as pl


def kernel(idx, table):
    raise NotImplementedError("write your pallas kernel here")


# speedup vs baseline: 3.3317x; 3.3317x over previous
import functools

import jax
import jax.numpy as jnp
from jax import lax
from jax.experimental import pallas as pl
from jax.experimental.pallas import tpu as pltpu
from jax.experimental.pallas import tpu_sc as plsc

FEATURE_DIM = 128
NUM_WORKERS = 32
CHUNK = 128


def _make_gather(n_rows):
    n_per_w = n_rows // NUM_WORKERS
    n_chunks = n_per_w // CHUNK

    mesh = plsc.VectorSubcoreMesh(core_axis_name="c", subcore_axis_name="s")

    @functools.partial(
        pl.kernel,
        mesh=mesh,
        out_type=jax.ShapeDtypeStruct((n_rows, FEATURE_DIM), jnp.float32),
        scratch_types=[
            pltpu.VMEM((n_chunks, CHUNK), jnp.int32),
            pltpu.VMEM((2, CHUNK, FEATURE_DIM), jnp.float32),
            pltpu.SemaphoreType.DMA,
            pltpu.SemaphoreType.DMA,
        ],
    )
    def gather_kernel(idx_hbm, table_hbm, out_hbm, idx_v, rows_v, sem0, sem1):
        wid = lax.axis_index("s") * 2 + lax.axis_index("c")
        base = wid * n_per_w
        pltpu.sync_copy(idx_hbm.at[wid], idx_v)

        sems = (sem0, sem1)

        def issue(j, slot):
            sem = sems[slot]
            pltpu.async_copy(table_hbm.at[idx_v.at[j]], rows_v.at[slot], sem)

        def wait_and_store(j, slot):
            sem = sems[slot]
            pltpu.make_async_copy(
                table_hbm.at[idx_v.at[j]], rows_v.at[slot], sem
            ).wait()
            pltpu.sync_copy(
                rows_v.at[slot], out_hbm.at[pl.ds(base + j * CHUNK, CHUNK)]
            )

        issue(0, 0)

        def body(g, carry):
            j0 = g * 2
            issue(j0 + 1, 1)
            wait_and_store(j0, 0)

            @pl.when(j0 + 2 < n_chunks)
            def _():
                issue(j0 + 2, 0)

            wait_and_store(j0 + 1, 1)
            return carry

        lax.fori_loop(0, n_chunks // 2, body, 0)

    return gather_kernel


def kernel(idx, table):
    b, s = idx.shape
    n_rows = b * s
    idx_flat = idx.reshape(
        NUM_WORKERS, n_rows // (NUM_WORKERS * CHUNK), CHUNK
    ).astype(jnp.int32)
    out = _make_gather(n_rows)(idx_flat, table)
    return out.reshape(b, s, FEATURE_DIM)


# speedup vs baseline: 3.3498x; 1.0054x over previous
import functools

import jax
import jax.numpy as jnp
from jax import lax
from jax.experimental import pallas as pl
from jax.experimental.pallas import tpu as pltpu
from jax.experimental.pallas import tpu_sc as plsc

FEATURE_DIM = 128
NUM_WORKERS = 32
CHUNK = 128
NBUF = 5


def _make_gather(n_rows):
    n_per_w = n_rows // NUM_WORKERS
    n_chunks = n_per_w // CHUNK
    n_groups = n_chunks // NBUF

    mesh = plsc.VectorSubcoreMesh(core_axis_name="c", subcore_axis_name="s")

    @functools.partial(
        pl.kernel,
        mesh=mesh,
        out_type=jax.ShapeDtypeStruct((n_rows, FEATURE_DIM), jnp.float32),
        scratch_types=[
            pltpu.VMEM((n_chunks, CHUNK), jnp.int32),
            pltpu.VMEM((NBUF, CHUNK, FEATURE_DIM), jnp.float32),
        ]
        + [pltpu.SemaphoreType.DMA] * (2 * NBUF),
    )
    def gather_kernel(idx_hbm, table_hbm, out_hbm, idx_v, rows_v, *sems):
        gsems = sems[:NBUF]
        osems = sems[NBUF:]
        wid = lax.axis_index("s") * 2 + lax.axis_index("c")
        base = wid * n_per_w
        pltpu.sync_copy(idx_hbm.at[wid], idx_v)

        def gather_cp(j, b):
            return pltpu.make_async_copy(
                table_hbm.at[idx_v.at[j]], rows_v.at[b], gsems[b]
            )

        def store_cp(j, b):
            return pltpu.make_async_copy(
                rows_v.at[b], out_hbm.at[pl.ds(base + j * CHUNK, CHUNK)],
                osems[b],
            )

        for b in range(NBUF):
            gather_cp(b, b).start()

        def body(g, carry):
            j0 = g * NBUF
            for b in range(NBUF):
                j = j0 + b
                gather_cp(j, b).wait()
                store_cp(j, b).start()

                @pl.when(j + NBUF < n_chunks)
                def _():
                    store_cp(j, b).wait()
                    gather_cp(j + NBUF, b).start()

            return carry

        lax.fori_loop(0, n_groups, body, 0)

        for b in range(NBUF):
            store_cp(n_chunks - NBUF + b, b).wait()

    return gather_kernel


def kernel(idx, table):
    b, s = idx.shape
    n_rows = b * s
    idx_flat = idx.reshape(
        NUM_WORKERS, n_rows // (NUM_WORKERS * CHUNK), CHUNK
    ).astype(jnp.int32)
    out = _make_gather(n_rows)(idx_flat, table)
    return out.reshape(b, s, FEATURE_DIM)


# speedup vs baseline: 5.9672x; 1.7814x over previous
import functools

import jax
import jax.numpy as jnp
from jax import lax
from jax.experimental import pallas as pl
from jax.experimental.pallas import tpu as pltpu
from jax.experimental.pallas import tpu_sc as plsc

FEATURE_DIM = 128
NUM_WORKERS = 32
NBUF = 4


def _make_gather(n_b, n_s):
    b_per_w = n_b // NUM_WORKERS

    mesh = plsc.VectorSubcoreMesh(core_axis_name="c", subcore_axis_name="s")

    @functools.partial(
        pl.kernel,
        mesh=mesh,
        out_type=jax.ShapeDtypeStruct((n_b, n_s, FEATURE_DIM), jnp.float32),
        scratch_types=[
            pltpu.VMEM((b_per_w, n_s), jnp.int32),
            pltpu.VMEM((NBUF, n_s, FEATURE_DIM), jnp.float32),
        ]
        + [pltpu.SemaphoreType.DMA] * (2 * NBUF),
        compiler_params=pltpu.CompilerParams(use_tc_tiling_on_sc=True),
    )
    def gather_kernel(idx_hbm, table_hbm, out_hbm, idx_v, rows_v, *sems):
        gsems = sems[:NBUF]
        osems = sems[NBUF:]
        wid = lax.axis_index("s") * 2 + lax.axis_index("c")
        base = wid * b_per_w
        pltpu.sync_copy(idx_hbm.at[pl.ds(base, b_per_w)], idx_v)

        def gather_cp(j, b):
            return pltpu.make_async_copy(
                table_hbm.at[idx_v.at[j]], rows_v.at[b], gsems[b]
            )

        def store_cp(j, b):
            return pltpu.make_async_copy(
                rows_v.at[b], out_hbm.at[base + j], osems[b]
            )

        for b in range(NBUF):
            gather_cp(b, b).start()

        def body(g, carry):
            j0 = g * NBUF
            for b in range(NBUF):
                j = j0 + b
                gather_cp(j, b).wait()
                store_cp(j, b).start()

                @pl.when(j + NBUF < b_per_w)
                def _():
                    store_cp(j, b).wait()
                    gather_cp(j + NBUF, b).start()

            return carry

        lax.fori_loop(0, b_per_w // NBUF, body, 0)

        for b in range(NBUF):
            store_cp(b_per_w - NBUF + b, b).wait()

    return gather_kernel


def kernel(idx, table):
    n_b, n_s = idx.shape
    return _make_gather(n_b, n_s)(idx.astype(jnp.int32), table)


# speedup vs baseline: 10.4493x; 1.7511x over previous
import functools

import jax
import jax.numpy as jnp
from jax import lax
from jax.experimental import pallas as pl
from jax.experimental.pallas import tpu as pltpu
from jax.experimental.pallas import tpu_sc as plsc

FEATURE_DIM = 128
NUM_WORKERS = 32
CHUNK = 128
NBUF = 5


def _make_gather(n_rows):
    n_per_w = n_rows // NUM_WORKERS
    n_chunks = n_per_w // CHUNK
    n_groups = n_chunks // NBUF

    mesh = plsc.VectorSubcoreMesh(core_axis_name="c", subcore_axis_name="s")

    @functools.partial(
        pl.kernel,
        mesh=mesh,
        out_type=jax.ShapeDtypeStruct((n_rows, FEATURE_DIM), jnp.float32),
        scratch_types=[
            pltpu.VMEM((n_chunks, CHUNK), jnp.int32),
            pltpu.VMEM((NBUF, CHUNK, FEATURE_DIM), jnp.float32),
        ]
        + [pltpu.SemaphoreType.DMA] * (2 * NBUF),
    )
    def gather_kernel(idx_hbm, table_hbm, out_hbm, idx_v, rows_v, *sems):
        gsems = sems[:NBUF]
        osems = sems[NBUF:]
        wid = lax.axis_index("s") * 2 + lax.axis_index("c")
        base = wid * n_per_w
        pltpu.sync_copy(idx_hbm.at[wid], idx_v)

        def gather_cp(j, b):
            return pltpu.make_async_copy(
                table_hbm.at[idx_v.at[j]], rows_v.at[b], gsems[b]
            )

        def store_cp(j, b):
            return pltpu.make_async_copy(
                rows_v.at[b], out_hbm.at[pl.ds(base + j * CHUNK, CHUNK)],
                osems[b],
            )

        for b in range(NBUF):
            gather_cp(b, b).start()

        def body(g, carry):
            j0 = g * NBUF
            for b in range(NBUF):
                j = j0 + b
                gather_cp(j, b).wait()
                store_cp(j, b).start()

                @pl.when(j + NBUF < n_chunks)
                def _():
                    store_cp(j, b).wait()
                    gather_cp(j + NBUF, b).start()

            return carry

        lax.fori_loop(0, n_groups, body, 0)

        for b in range(NBUF):
            store_cp(n_chunks - NBUF + b, b).wait()

    return gather_kernel


def kernel(idx, table):
    n_b, n_s = idx.shape
    n_rows = n_b * n_s
    idx_t = idx.T.astype(jnp.int32).reshape(
        NUM_WORKERS, n_rows // (NUM_WORKERS * CHUNK), CHUNK
    )
    out = _make_gather(n_rows)(idx_t, table)
    return out.reshape(n_s, n_b, FEATURE_DIM).transpose(1, 0, 2)


# speedup vs baseline: 10.4676x; 1.0018x over previous
import functools

import jax
import jax.numpy as jnp
from jax import lax
from jax.experimental import pallas as pl
from jax.experimental.pallas import tpu as pltpu
from jax.experimental.pallas import tpu_sc as plsc

FEATURE_DIM = 128
NUM_WORKERS = 32
CHUNK = 64
NBUF = 10


def _make_gather(n_rows):
    n_per_w = n_rows // NUM_WORKERS
    n_chunks = n_per_w // CHUNK
    n_groups = n_chunks // NBUF

    mesh = plsc.VectorSubcoreMesh(core_axis_name="c", subcore_axis_name="s")

    @functools.partial(
        pl.kernel,
        mesh=mesh,
        out_type=jax.ShapeDtypeStruct((n_rows, FEATURE_DIM), jnp.float32),
        scratch_types=[
            pltpu.VMEM((n_chunks, CHUNK), jnp.int32),
            pltpu.VMEM((NBUF, CHUNK, FEATURE_DIM), jnp.float32),
        ]
        + [pltpu.SemaphoreType.DMA] * (2 * NBUF),
    )
    def gather_kernel(idx_hbm, table_hbm, out_hbm, idx_v, rows_v, *sems):
        gsems = sems[:NBUF]
        osems = sems[NBUF:]
        wid = lax.axis_index("s") * 2 + lax.axis_index("c")
        base = wid * n_per_w
        pltpu.sync_copy(idx_hbm.at[wid], idx_v)

        def gather_cp(j, b):
            return pltpu.make_async_copy(
                table_hbm.at[idx_v.at[j]], rows_v.at[b], gsems[b]
            )

        def store_cp(j, b):
            return pltpu.make_async_copy(
                rows_v.at[b], out_hbm.at[pl.ds(base + j * CHUNK, CHUNK)],
                osems[b],
            )

        for b in range(NBUF):
            gather_cp(b, b).start()

        def body(g, carry):
            j0 = g * NBUF
            for b in range(NBUF):
                j = j0 + b
                gather_cp(j, b).wait()
                store_cp(j, b).start()

                @pl.when(j + NBUF < n_chunks)
                def _():
                    store_cp(j, b).wait()
                    gather_cp(j + NBUF, b).start()

            return carry

        lax.fori_loop(0, n_groups, body, 0)

        for b in range(NBUF):
            store_cp(n_chunks - NBUF + b, b).wait()

    return gather_kernel


def kernel(idx, table):
    n_b, n_s = idx.shape
    n_rows = n_b * n_s
    idx_t = idx.T.astype(jnp.int32).reshape(
        NUM_WORKERS, n_rows // (NUM_WORKERS * CHUNK), CHUNK
    )
    out = _make_gather(n_rows)(idx_t, table)
    return out.reshape(n_s, n_b, FEATURE_DIM).transpose(1, 0, 2)
